# Initial kernel scaffold; baseline (speedup 1.0000x reference)
import jax
import jax.numpy as jnp
from jax.experimental import pallas as pl


def kernel(user_id, movie_id, movie_categories, emb_users, emb_movies, emb_movie_cats, bias_user, bias_movie):
    raise NotImplementedError("write your pallas kernel here")


# speedup vs baseline: 2.6710x; 2.6710x over previous
import functools

import jax
import jax.numpy as jnp
from jax import lax
from jax.experimental import pallas as pl
from jax.experimental.pallas import tpu as pltpu
from jax.experimental.pallas import tpu_sc as plsc

NUM_USERS = 1000000
NUM_MOVIES = 100000
NUM_CATS = 1000
U_DIM = 64
M_DIM = 32
C_DIM = 32
B = 16384
L = 20
MARGIN = 0.1

_INFO = plsc.get_sparse_core_info()
NC = _INFO.num_cores
NS = _INFO.num_subcores
LANES = _INFO.num_lanes
NW = NC * NS
RPW = B // NW


@functools.partial(
    pl.kernel,
    out_type=jax.ShapeDtypeStruct((B,), jnp.float32),
    mesh=plsc.VectorSubcoreMesh(core_axis_name="c", subcore_axis_name="s"),
    compiler_params=pltpu.CompilerParams(
        needs_layout_passes=False, use_tc_tiling_on_sc=False),
    scratch_types=[
        pltpu.VMEM((RPW,), jnp.int32),
        pltpu.VMEM((RPW,), jnp.int32),
        pltpu.VMEM((RPW, L), jnp.int32),
        pltpu.VMEM((RPW, U_DIM), jnp.float32),
        pltpu.VMEM((RPW, M_DIM), jnp.float32),
        pltpu.VMEM((NUM_CATS, C_DIM), jnp.float32),
        pltpu.VMEM((RPW,), jnp.float32),
        pltpu.VMEM((RPW,), jnp.float32),
        pltpu.VMEM((RPW,), jnp.float32),
        pltpu.SemaphoreType.DMA,
    ],
)
def _sc_forward(uid_hbm, mid_hbm, cidx_hbm, users_hbm, movies_hbm, ctab_hbm,
                bu_hbm, bm_hbm, out_hbm,
                uid_v, mid_v, cidx_v, u_rows, m_rows, ctab_v, bu_v, bm_v,
                res_v, sem):
    wid = lax.axis_index("s") * NC + lax.axis_index("c")
    base = wid * RPW

    pltpu.sync_copy(uid_hbm.at[pl.ds(base, RPW)], uid_v)
    pltpu.sync_copy(mid_hbm.at[pl.ds(base, RPW)], mid_v)
    pltpu.sync_copy(cidx_hbm.at[pl.ds(base, RPW)], cidx_v)
    pltpu.sync_copy(ctab_hbm, ctab_v)
    pltpu.async_copy(users_hbm.at[uid_v], u_rows, sem).wait()
    pltpu.async_copy(movies_hbm.at[mid_v], m_rows, sem).wait()
    pltpu.async_copy(bu_hbm.at[uid_v], bu_v, sem).wait()
    pltpu.async_copy(bm_hbm.at[mid_v], bm_v, sem).wait()

    iota = lax.iota(jnp.int32, LANES)

    dnums = lax.GatherDimensionNumbers(
        offset_dims=(), collapsed_slice_dims=(0,), start_index_map=(0,))

    def lane_shuffle(x, idx):
        return lax.gather(x, idx[:, None], dnums, slice_sizes=(1,),
                          mode=lax.GatherScatterMode.PROMISE_IN_BOUNDS)

    def hsum(x):
        for sft in (8, 4, 2, 1):
            x = x + lane_shuffle(x, iota ^ sft)
        return x

    def row_result(r):
        idx_a = cidx_v[r, pl.ds(0, LANES)]
        idx_b = cidx_v[r, pl.ds(L - LANES, LANES)]
        tail = iota >= (2 * LANES - L)
        cntf = hsum(
            jnp.where(idx_a != 0, 1.0, 0.0)
            + jnp.where(jnp.logical_and(tail, idx_b != 0), 1.0, 0.0))
        acc0 = jnp.zeros((LANES,), jnp.float32)
        acc1 = jnp.zeros((LANES,), jnp.float32)
        for sl in range(LANES):
            cvec = jnp.full((LANES,), idx_a[sl], jnp.int32)
            acc0 = acc0 + plsc.load_gather(ctab_v, [cvec, iota])
            acc1 = acc1 + plsc.load_gather(ctab_v, [cvec, iota + LANES])
        for sl in range(2 * LANES - L, LANES):
            cvec = jnp.full((LANES,), idx_b[sl], jnp.int32)
            acc0 = acc0 + plsc.load_gather(ctab_v, [cvec, iota])
            acc1 = acc1 + plsc.load_gather(ctab_v, [cvec, iota + LANES])
        u0 = u_rows[r, pl.ds(0, LANES)]
        u1 = u_rows[r, pl.ds(LANES, LANES)]
        u2 = u_rows[r, pl.ds(2 * LANES, LANES)]
        u3 = u_rows[r, pl.ds(3 * LANES, LANES)]
        m0 = m_rows[r, pl.ds(0, LANES)]
        m1 = m_rows[r, pl.ds(LANES, LANES)]
        inv = 1.0 / jnp.maximum(cntf, 1.0)
        s = u0 * m0 + u1 * m1 + (u2 * acc0 + u3 * acc1) * inv
        return hsum(s)

    def grp_compute(g, carry):
        def lane_body(j, part):
            p = row_result(g * LANES + j)
            return jnp.where(iota == j, p, part)
        part = lax.fori_loop(0, LANES, lane_body,
                             jnp.zeros((LANES,), jnp.float32))
        res_v[pl.ds(g * LANES, LANES)] = part
        return carry

    lax.fori_loop(0, RPW // LANES, grp_compute, 0)

    def grp_body(g, carry):
        off = g * LANES
        x = (res_v[pl.ds(off, LANES)] + bu_v[pl.ds(off, LANES)]
             + bm_v[pl.ds(off, LANES)])
        sig = 1.0 / (1.0 + jnp.exp(-x))
        res_v[pl.ds(off, LANES)] = sig * (1.0 + 2 * MARGIN) - MARGIN
        return carry

    lax.fori_loop(0, RPW // LANES, grp_body, 0)
    pltpu.sync_copy(res_v, out_hbm.at[pl.ds(base, RPW)])


def kernel(user_id, movie_id, movie_categories, emb_users, emb_movies,
           emb_movie_cats, bias_user, bias_movie):
    uid = user_id.astype(jnp.int32)
    mid = movie_id.astype(jnp.int32)
    cidx = movie_categories.astype(jnp.int32)
    bu = bias_user.reshape(-1).astype(jnp.float32)
    bm = bias_movie.reshape(-1).astype(jnp.float32)
    return _sc_forward(uid, mid, cidx, emb_users, emb_movies,
                       emb_movie_cats, bu, bm)
